# baseline (speedup 1.0000x reference)
import functools

import jax
import jax.numpy as jnp
from jax import lax
from jax.experimental import pallas as pl
from jax.experimental.pallas import tpu as pltpu
from jax.experimental.pallas import tpu_sc as plsc

_L = 16


def _interp_round_index(t, av_ref, T):
    lanes = lax.iota(jnp.int32, _L)
    ones = jnp.ones((_L,), jnp.int32)
    zeros = jnp.zeros((_L,), jnp.int32)
    cnt = jnp.int32(0)
    for c in range(T // _L):
        xs = av_ref[pl.ds(c * _L, _L)]
        cnt = cnt + lax.reduce_sum_p.bind(
            jnp.where(xs <= t, ones, zeros), axes=(0,))
    j = jnp.clip(cnt - 1, 0, T - 2)
    j_vec = jnp.full((_L,), j, jnp.int32)
    xj = plsc.load_gather(av_ref, [j_vec])
    xj1 = plsc.load_gather(av_ref, [j_vec + 1])
    jf = j_vec.astype(jnp.float32)
    approx = jf + (t - xj) / (xj1 - xj)
    approx = jnp.clip(approx, 0.0, float(T - 1))
    rcnt = jnp.int32(0)
    ties_even = jnp.int32(0)
    for c in range(T // _L):
        k = lanes + c * _L
        h = k.astype(jnp.float32) + 0.5
        rcnt = rcnt + lax.reduce_sum_p.bind(
            jnp.where(approx >= h, ones, zeros), axes=(0,))
        tie = (approx == h) & ((k & 1) == 0)
        ties_even = ties_even + lax.reduce_sum_p.bind(
            jnp.where(tie, ones, zeros), axes=(0,))
    return rcnt - ties_even


def _make_sc_kernel(T, H, W, dtype):
    rows = H // 32
    sub = rows // 2

    mesh = plsc.VectorSubcoreMesh(core_axis_name="c", subcore_axis_name="s")

    @functools.partial(
        pl.kernel,
        mesh=mesh,
        compiler_params=pltpu.CompilerParams(needs_layout_passes=False),
        out_type=(
            jax.ShapeDtypeStruct((H, W), dtype),
            jax.ShapeDtypeStruct((H, W), dtype),
        ),
        scratch_types=[
            pltpu.VMEM((_L,), jnp.float32),
            pltpu.VMEM((T,), jnp.float32),
            pltpu.VMEM((2, sub, W), dtype),
            pltpu.VMEM((2, sub, W), dtype),
            pltpu.SemaphoreType.DMA,
            pltpu.SemaphoreType.DMA,
            pltpu.SemaphoreType.DMA,
            pltpu.SemaphoreType.DMA,
        ],
    )
    def sc_slice(time_hbm, av_hbm, temp_hbm, geo_hbm, out_t_hbm, out_g_hbm,
                 t_v, av_v, buf_t, buf_g, s0, s1, s2, s3):
        cid = lax.axis_index("c")
        sid = lax.axis_index("s")
        ct = pltpu.async_copy(time_hbm, t_v.at[pl.ds(0, 1)], s0)
        ca = pltpu.async_copy(av_hbm, av_v, s1)
        ct.wait()
        ca.wait()
        t = jnp.full((_L,), t_v[...][0], jnp.float32)
        idx = _interp_round_index(t, av_v, T)
        wid = sid * 2 + cid
        base = wid * rows
        g0 = pltpu.async_copy(temp_hbm.at[idx, pl.ds(base, sub)],
                              buf_t.at[0], s0)
        g1 = pltpu.async_copy(temp_hbm.at[idx, pl.ds(base + sub, sub)],
                              buf_t.at[1], s1)
        g2 = pltpu.async_copy(geo_hbm.at[idx, pl.ds(base, sub)],
                              buf_g.at[0], s2)
        g3 = pltpu.async_copy(geo_hbm.at[idx, pl.ds(base + sub, sub)],
                              buf_g.at[1], s3)
        g0.wait()
        c0 = pltpu.async_copy(buf_t.at[0], out_t_hbm.at[pl.ds(base, sub)], s0)
        g1.wait()
        c1 = pltpu.async_copy(buf_t.at[1],
                              out_t_hbm.at[pl.ds(base + sub, sub)], s1)
        g2.wait()
        c2 = pltpu.async_copy(buf_g.at[0], out_g_hbm.at[pl.ds(base, sub)], s2)
        g3.wait()
        c3 = pltpu.async_copy(buf_g.at[1],
                              out_g_hbm.at[pl.ds(base + sub, sub)], s3)
        c0.wait()
        c1.wait()
        c2.wait()
        c3.wait()

    return sc_slice


def kernel(time, available_time, temperature, geopotential):
    T = available_time.shape[0]
    H, W = temperature.shape[1], temperature.shape[2]
    sc = _make_sc_kernel(T, H, W, temperature.dtype)
    out_t, out_g = sc(time.astype(jnp.float32),
                      available_time.astype(jnp.float32),
                      temperature, geopotential)
    return (out_t, out_g)
